# baseline (speedup 1.0000x reference)
import functools

import jax
import jax.numpy as jnp
from jax import lax
from jax.experimental import pallas as pl
from jax.experimental.pallas import tpu as pltpu
from jax.experimental.pallas import tpu_sc as plsc

N = 10000
D = 128
E = 320000
L = 3
BN_EPS = 1e-5

NC = 2
NS = 16
NW = NC * NS
K = 104
NB = 3
EPW = E // NW
CF = EPW // K
REM = EPW - CF * K
N_PAD = 10240
ZB = 32
ZN = N_PAD // NS // ZB


_mesh = plsc.VectorSubcoreMesh(core_axis_name="c", subcore_axis_name="s")


@functools.partial(
    pl.kernel,
    out_type=jax.ShapeDtypeStruct((NC, N_PAD, D), jnp.float32),
    mesh=_mesh,
    scratch_types=[
        pltpu.VMEM((NB, K), jnp.int32),
        pltpu.VMEM((NB, K), jnp.int32),
        pltpu.VMEM((NB, K, D), jnp.float32),
        pltpu.VMEM((ZB, D), jnp.float32),
        pltpu.VMEM((REM,), jnp.int32),
        pltpu.VMEM((REM,), jnp.int32),
        pltpu.VMEM((REM, D), jnp.float32),
        pltpu.VMEM_SHARED((N_PAD, D), jnp.float32),
        pltpu.SemaphoreType.DMA,
        pltpu.SemaphoreType.DMA,
        pltpu.SemaphoreType.DMA,
        pltpu.SemaphoreType.DMA,
        pltpu.SemaphoreType.DMA,
        pltpu.SemaphoreType.DMA,
        pltpu.SemaphoreType.DMA,
        pltpu.SemaphoreType.DMA,
        pltpu.SemaphoreType.DMA,
    ],
)
def _edge_pass(r_hbm, src_hbm, dst_hbm, out_hbm, srcb, dstb, rows2, zbuf, srcs,
               dsts, rowss, agg_sh, gsem0, gsem1, gsem2, gsem3, isem0, isem1,
               isem2, isem3, zsem):
    cid = lax.axis_index("c")
    sid = lax.axis_index("s")
    wid = cid * NS + sid
    gsems = (gsem0, gsem1, gsem2, gsem3)
    isems = (isem0, isem1, isem2, isem3)
    base0 = wid * EPW

    def _idx_copies(g, b):
        base = base0 + g * K
        return (
            pltpu.make_async_copy(src_hbm.at[pl.ds(base, K)], srcb.at[b],
                                  isems[b]),
            pltpu.make_async_copy(dst_hbm.at[pl.ds(base, K)], dstb.at[b],
                                  isems[b]),
        )

    def _fire_idx(g, b):
        for c in _idx_copies(g, b):
            c.start()

    def _drain_idx(g, b):
        for c in _idx_copies(g, b):
            c.wait()

    def _gather(g, b):
        return pltpu.make_async_copy(r_hbm.at[srcb.at[b]], rows2.at[b], gsems[b])

    def _scat(b):
        pltpu.sync_copy(rows2.at[b], agg_sh.at[dstb.at[b]], add=True)

    for b in range(NB):
        _fire_idx(b, b)
    for b in range(NB - 1):
        _drain_idx(b, b)
        _gather(b, b).start()

    def _zrow(i, carry):
        for j in range(D // 16):
            zbuf[i, pl.ds(j * 16, 16)] = jnp.zeros((16,), jnp.float32)
        return carry

    lax.fori_loop(0, ZB, _zrow, 0)
    zcopies = [
        pltpu.make_async_copy(
            zbuf, agg_sh.at[pl.ds(sid * (N_PAD // NS) + z * ZB, ZB)], zsem)
        for z in range(ZN)
    ]
    for c in zcopies:
        c.start()
    for c in zcopies:
        c.wait()
    plsc.subcore_barrier()

    def _group(t, carry):
        g0 = t * NB
        for b in range(NB):
            g = g0 + b
            b3 = (b + NB - 1) % NB
            _drain_idx(g + NB - 1, b3)
            _gather(g + NB - 1, b3).start()
            _gather(g, b).wait()
            _scat(b)
            _fire_idx(g + NB, b)
        return carry

    lax.fori_loop(0, CF // NB - 1, _group, 0)

    rem_base = base0 + CF * K
    rem_copies = (
        pltpu.make_async_copy(src_hbm.at[pl.ds(rem_base, REM)], srcs, isem0),
        pltpu.make_async_copy(dst_hbm.at[pl.ds(rem_base, REM)], dsts, isem0),
    )
    rem_gather = pltpu.make_async_copy(r_hbm.at[srcs], rowss, gsem0)
    g0 = CF - NB
    _drain_idx(g0 + NB - 1, NB - 1)
    _gather(g0 + NB - 1, NB - 1).start()
    _gather(g0, 0).wait()
    _scat(0)
    for c in rem_copies:
        c.start()
    _gather(g0 + 1, 1).wait()
    _scat(1)
    for c in rem_copies:
        c.wait()
    rem_gather.start()
    for b in range(2, NB):
        _gather(g0 + b, b).wait()
        _scat(b)
    rem_gather.wait()
    pltpu.sync_copy(rowss, agg_sh.at[dsts], add=True)
    plsc.subcore_barrier()

    off = sid * (N_PAD // NS)
    pltpu.sync_copy(agg_sh.at[pl.ds(off, N_PAD // NS)],
                    out_hbm.at[cid, pl.ds(off, N_PAD // NS)])


def _prep_body(x_ref, r_ref):
    r_ref[...] = jnp.maximum(x_ref[...], 0.0)


_prep = pl.pallas_call(
    _prep_body,
    out_shape=jax.ShapeDtypeStruct((N, D), jnp.float32),
)


def _bn(v, g, b):
    mean = jnp.sum(v, axis=0, keepdims=True) * (1.0 / N)
    sq = jnp.sum(v * v, axis=0, keepdims=True) * (1.0 / N)
    inv = lax.rsqrt(jnp.maximum(sq - mean * mean, 0.0) + BN_EPS) * g
    return v * inv + (b - mean * inv)


def _mlp_body(l, relu_out, h_ref, p_ref, w1_ref, b1_ref, g1_ref, bt1_ref,
              w2_ref, b2_ref, go_ref, bo_ref, eps_ref, h_out, r_out):
    agg = p_ref[0, :N, :] + p_ref[1, :N, :]
    pre = (1.0 + eps_ref[0, l]) * h_ref[...] + agg
    hid = jnp.dot(pre, w1_ref[l], preferred_element_type=jnp.float32) + b1_ref[l]
    hid = jnp.maximum(_bn(hid, g1_ref[l], bt1_ref[l]), 0.0)
    out = jnp.dot(hid, w2_ref[l], preferred_element_type=jnp.float32) + b2_ref[l]
    out = _bn(out, go_ref[l], bo_ref[l])
    if relu_out:
        out = jnp.maximum(out, 0.0)
    h_out[...] = out
    r_out[...] = jnp.maximum(out, 0.0)


def _make_mlp(l, relu_out):
    return pl.pallas_call(
        functools.partial(_mlp_body, l, relu_out),
        in_specs=[pl.BlockSpec()] * 10 + [pl.BlockSpec(memory_space=pltpu.SMEM)],
        out_shape=(
            jax.ShapeDtypeStruct((N, D), jnp.float32),
            jax.ShapeDtypeStruct((N, D), jnp.float32),
        ),
    )


_mlps = [_make_mlp(l, l < L - 1) for l in range(L)]


def kernel(x, edge_index, W1, b1, g1, bt1, W2, b2, eps, g_out, b_out):
    eps2d = eps.reshape(1, L)
    h = x
    r = _prep(x)
    for l in range(L):
        parts = _edge_pass(r, edge_index[0], edge_index[1])
        h, r = _mlps[l](h, parts, W1, b1, g1, bt1, W2, b2, g_out, b_out, eps2d)
    return h


# speedup vs baseline: 1.0082x; 1.0082x over previous
import functools

import jax
import jax.numpy as jnp
from jax import lax
from jax.experimental import pallas as pl
from jax.experimental.pallas import tpu as pltpu
from jax.experimental.pallas import tpu_sc as plsc

N = 10000
D = 128
E = 320000
L = 3
BN_EPS = 1e-5

NC = 2
NS = 16
NW = NC * NS
K = 128
NB = 2
EPW = E // NW
CF = EPW // K
REM = EPW - CF * K
N_PAD = 10240
ZB = 32
ZN = N_PAD // NS // ZB


_mesh = plsc.VectorSubcoreMesh(core_axis_name="c", subcore_axis_name="s")


@functools.partial(
    pl.kernel,
    out_type=jax.ShapeDtypeStruct((NC, N_PAD, D), jnp.float32),
    mesh=_mesh,
    scratch_types=[
        pltpu.VMEM((NB, K), jnp.int32),
        pltpu.VMEM((NB, K), jnp.int32),
        pltpu.VMEM((NB, K, D), jnp.float32),
        pltpu.VMEM((ZB, D), jnp.float32),
        pltpu.VMEM((REM,), jnp.int32),
        pltpu.VMEM((REM,), jnp.int32),
        pltpu.VMEM((REM, D), jnp.float32),
        pltpu.VMEM_SHARED((N_PAD, D), jnp.float32),
        pltpu.SemaphoreType.DMA,
        pltpu.SemaphoreType.DMA,
        pltpu.SemaphoreType.DMA,
        pltpu.SemaphoreType.DMA,
        pltpu.SemaphoreType.DMA,
        pltpu.SemaphoreType.DMA,
        pltpu.SemaphoreType.DMA,
        pltpu.SemaphoreType.DMA,
        pltpu.SemaphoreType.DMA,
    ],
)
def _edge_pass(r_hbm, src_hbm, dst_hbm, out_hbm, srcb, dstb, rows2, zbuf, srcs,
               dsts, rowss, agg_sh, gsem0, gsem1, gsem2, gsem3, isem0, isem1,
               isem2, isem3, zsem):
    cid = lax.axis_index("c")
    sid = lax.axis_index("s")
    wid = cid * NS + sid
    gsems = (gsem0, gsem1, gsem2, gsem3)
    isems = (isem0, isem1, isem2, isem3)
    base0 = wid * EPW

    def _idx_copies(g, b):
        base = base0 + g * K
        return (
            pltpu.make_async_copy(src_hbm.at[pl.ds(base, K)], srcb.at[b],
                                  isems[b]),
            pltpu.make_async_copy(dst_hbm.at[pl.ds(base, K)], dstb.at[b],
                                  isems[b]),
        )

    def _fire_idx(g, b):
        for c in _idx_copies(g, b):
            c.start()

    def _drain_idx(g, b):
        for c in _idx_copies(g, b):
            c.wait()

    def _gather(g, b):
        return pltpu.make_async_copy(r_hbm.at[srcb.at[b]], rows2.at[b], gsems[b])

    def _scat(b):
        pltpu.sync_copy(rows2.at[b], agg_sh.at[dstb.at[b]], add=True)

    for b in range(NB):
        _fire_idx(b, b)
    for b in range(NB - 1):
        _drain_idx(b, b)
        _gather(b, b).start()

    def _zrow(i, carry):
        for j in range(D // 16):
            zbuf[i, pl.ds(j * 16, 16)] = jnp.zeros((16,), jnp.float32)
        return carry

    lax.fori_loop(0, ZB, _zrow, 0)
    zcopies = [
        pltpu.make_async_copy(
            zbuf, agg_sh.at[pl.ds(sid * (N_PAD // NS) + z * ZB, ZB)], zsem)
        for z in range(ZN)
    ]
    for c in zcopies:
        c.start()
    for c in zcopies:
        c.wait()
    plsc.subcore_barrier()

    def _group(t, carry):
        g0 = t * NB
        for b in range(NB):
            g = g0 + b
            b3 = (b + NB - 1) % NB
            _drain_idx(g + NB - 1, b3)
            _gather(g + NB - 1, b3).start()
            _gather(g, b).wait()
            _scat(b)
            _fire_idx(g + NB, b)
        return carry

    lax.fori_loop(0, CF // NB - 1, _group, 0)

    rem_base = base0 + CF * K
    rem_copies = (
        pltpu.make_async_copy(src_hbm.at[pl.ds(rem_base, REM)], srcs, isem0),
        pltpu.make_async_copy(dst_hbm.at[pl.ds(rem_base, REM)], dsts, isem0),
    )
    rem_gather = pltpu.make_async_copy(r_hbm.at[srcs], rowss, gsem0)
    g0 = CF - NB
    _drain_idx(g0 + NB - 1, NB - 1)
    _gather(g0 + NB - 1, NB - 1).start()
    _gather(g0, 0).wait()
    _scat(0)
    for c in rem_copies:
        c.start()
    _gather(g0 + 1, 1).wait()
    _scat(1)
    for c in rem_copies:
        c.wait()
    rem_gather.start()
    for b in range(2, NB):
        _gather(g0 + b, b).wait()
        _scat(b)
    rem_gather.wait()
    pltpu.sync_copy(rowss, agg_sh.at[dsts], add=True)
    plsc.subcore_barrier()

    off = sid * (N_PAD // NS)
    pltpu.sync_copy(agg_sh.at[pl.ds(off, N_PAD // NS)],
                    out_hbm.at[cid, pl.ds(off, N_PAD // NS)])


def _prep_body(x_ref, r_ref):
    r_ref[...] = jnp.maximum(x_ref[...], 0.0)


_prep = pl.pallas_call(
    _prep_body,
    out_shape=jax.ShapeDtypeStruct((N, D), jnp.float32),
)


def _bn(v, g, b):
    mean = jnp.sum(v, axis=0, keepdims=True) * (1.0 / N)
    sq = jnp.sum(v * v, axis=0, keepdims=True) * (1.0 / N)
    inv = lax.rsqrt(jnp.maximum(sq - mean * mean, 0.0) + BN_EPS) * g
    return v * inv + (b - mean * inv)


def _mlp_body(l, relu_out, h_ref, p_ref, w1_ref, b1_ref, g1_ref, bt1_ref,
              w2_ref, b2_ref, go_ref, bo_ref, eps_ref, h_out, r_out):
    agg = p_ref[0, :N, :] + p_ref[1, :N, :]
    pre = (1.0 + eps_ref[0, l]) * h_ref[...] + agg
    hid = jnp.dot(pre, w1_ref[l], preferred_element_type=jnp.float32) + b1_ref[l]
    hid = jnp.maximum(_bn(hid, g1_ref[l], bt1_ref[l]), 0.0)
    out = jnp.dot(hid, w2_ref[l], preferred_element_type=jnp.float32) + b2_ref[l]
    out = _bn(out, go_ref[l], bo_ref[l])
    if relu_out:
        out = jnp.maximum(out, 0.0)
    h_out[...] = out
    r_out[...] = jnp.maximum(out, 0.0)


def _make_mlp(l, relu_out):
    return pl.pallas_call(
        functools.partial(_mlp_body, l, relu_out),
        in_specs=[pl.BlockSpec()] * 10 + [pl.BlockSpec(memory_space=pltpu.SMEM)],
        out_shape=(
            jax.ShapeDtypeStruct((N, D), jnp.float32),
            jax.ShapeDtypeStruct((N, D), jnp.float32),
        ),
    )


_mlps = [_make_mlp(l, l < L - 1) for l in range(L)]


def kernel(x, edge_index, W1, b1, g1, bt1, W2, b2, eps, g_out, b_out):
    eps2d = eps.reshape(1, L)
    h = x
    r = _prep(x)
    for l in range(L):
        parts = _edge_pass(r, edge_index[0], edge_index[1])
        h, r = _mlps[l](h, parts, W1, b1, g1, bt1, W2, b2, g_out, b_out, eps2d)
    return h
